# baseline (speedup 1.0000x reference)
import functools

import jax
import jax.numpy as jnp
from jax import lax
from jax.experimental import pallas as pl
from jax.experimental.pallas import tpu as pltpu
from jax.experimental.pallas import tpu_sc as plsc

_SELFW = 1.0
_NC = 2
_NS = 16
_CHUNK = 120
_BLK = 3
_NBUF = 3


@functools.lru_cache(maxsize=None)
def _build_sc_agg(n, din, nblk, acc_rows):
    rows_per_tile = acc_rows // _NS
    assert rows_per_tile * _NS == acc_rows and nblk % 2 == 0
    assert (4 * _BLK) % _NBUF == 0
    nu = nblk // 2

    mesh = plsc.VectorSubcoreMesh(core_axis_name="c", subcore_axis_name="s")

    @functools.partial(
        pl.kernel,
        mesh=mesh,
        out_type=jax.ShapeDtypeStruct((_NC, acc_rows, din), jnp.float32),
        scratch_types=[
            pltpu.VMEM_SHARED((acc_rows, din), jnp.float32),
            pltpu.VMEM((_CHUNK, din), jnp.float32),
            pltpu.VMEM((_CHUNK, din), jnp.float32),
            pltpu.VMEM((_CHUNK, din), jnp.float32),
            pltpu.VMEM((2, _BLK, _CHUNK), jnp.int32),
            pltpu.VMEM((2, _BLK, _CHUNK), jnp.int32),
            pltpu.SemaphoreType.DMA,
            pltpu.SemaphoreType.DMA,
            pltpu.SemaphoreType.DMA,
            pltpu.SemaphoreType.DMA,
            pltpu.SemaphoreType.DMA,
        ],
    )
    def agg(x_hbm, idx_hbm, parts_hbm, acc, rows0, rows1, rows2, ib_a, ib_b,
            sem0, sem1, sem2, isem_a, isem_b):
        c = lax.axis_index("c")
        s = lax.axis_index("s")
        wid = s * _NC + c
        rows = (rows0, rows1, rows2)
        sems = (sem0, sem1, sem2)

        zeros16 = jnp.zeros((16,), jnp.float32)

        def zbody(i, carry):
            for j in range(din // 16):
                rows0[i, pl.ds(j * 16, 16)] = zeros16
            return carry

        lax.fori_loop(0, _CHUNK, zbody, 0)
        nfull, rem = divmod(rows_per_tile, _CHUNK)
        for j in range(nfull):
            r0 = s * rows_per_tile + j * _CHUNK
            pltpu.sync_copy(rows0, acc.at[pl.ds(r0, _CHUNK)])
        if rem:
            r0 = s * rows_per_tile + nfull * _CHUNK
            pltpu.sync_copy(rows0.at[pl.ds(0, rem)], acc.at[pl.ds(r0, rem)])
        plsc.subcore_barrier()

        def gstart(ib, v, g):
            j, d = divmod(v, 2)
            pltpu.async_copy(x_hbm.at[ib.at[d, j]], rows[g % _NBUF],
                             sems[g % _NBUF])

        def gwait(ib, v, g):
            j, d = divmod(v, 2)
            pltpu.make_async_copy(x_hbm.at[ib.at[d, j]], rows[g % _NBUF],
                                  sems[g % _NBUF]).wait()

        pltpu.sync_copy(idx_hbm.at[:, wid, 0], ib_a)
        pltpu.async_copy(idx_hbm.at[:, wid, 1], ib_b, isem_b)
        gstart(ib_a, 0, 0)
        gstart(ib_a, 1, 1)

        nv = 2 * _BLK

        def do_block(u, ib, nxt, nxt_isem, guarded):
            base = nv if guarded else 0
            for v in range(nv):
                gwait(ib, v, base + v)
                t = v + 2
                if t < nv:
                    gstart(ib, t, base + t)
                else:
                    def rollover(t=t):
                        if t == nv:
                            pltpu.make_async_copy(idx_hbm.at[:, wid, 0], nxt,
                                                  nxt_isem).wait()
                        gstart(nxt, t - nv, base + t)

                    if guarded:
                        pl.when(u < nu - 1)(rollover)
                    else:
                        rollover()
                j, d = divmod(v, 2)
                pltpu.sync_copy(rows[(base + v) % _NBUF], acc.at[ib.at[1 - d, j]],
                                add=True)

        def ebody(u, carry):
            do_block(u, ib_a, ib_b, isem_b, False)

            @pl.when(u < nu - 1)
            def _():
                pltpu.async_copy(idx_hbm.at[:, wid, 2 * u + 2], ib_a, isem_a)

            do_block(u, ib_b, ib_a, isem_a, True)

            @pl.when(u < nu - 1)
            def _():
                pltpu.async_copy(idx_hbm.at[:, wid, 2 * u + 3], ib_b, isem_b)

            return carry

        lax.fori_loop(0, nu, ebody, 0)
        plsc.subcore_barrier()

        for j in range(nfull):
            r0 = s * rows_per_tile + j * _CHUNK
            pltpu.sync_copy(acc.at[pl.ds(r0, _CHUNK)], rows0)
            pltpu.sync_copy(rows0, parts_hbm.at[c, pl.ds(r0, _CHUNK)])
        if rem:
            r0 = s * rows_per_tile + nfull * _CHUNK
            pltpu.sync_copy(acc.at[pl.ds(r0, rem)], rows0.at[pl.ds(0, rem)])
            pltpu.sync_copy(rows0.at[pl.ds(0, rem)],
                            parts_hbm.at[c, pl.ds(r0, rem)])

    return agg


def _make_mm1_body(bm, pad, n):
    full = pad // n
    rem = pad % n

    def _mm1_body(x_ref, w_ref, b_ref, o_ref):
        rowid = bm * pl.program_id(0) + jax.lax.broadcasted_iota(
            jnp.int32, (bm, 1), 0)
        cnt = full + (rowid < rem).astype(jnp.float32)
        y = (_SELFW - 2.0 * cnt) * x_ref[...]
        o_ref[...] = jnp.dot(
            y, w_ref[...], preferred_element_type=jnp.float32
        ) + (2.0 + _SELFW) * b_ref[...]

    return _mm1_body


def _mm2_body(s_ref, p_ref, w_ref, o_ref):
    o_ref[...] = s_ref[...] + jnp.dot(
        p_ref[0] + p_ref[1], w_ref[...], preferred_element_type=jnp.float32)


def kernel(x, edge_index, W, b):
    n, din = x.shape
    dout = W.shape[1]
    e = edge_index.shape[1]
    nw = _NC * _NS

    iter_edges = 2 * _BLK * _CHUNK
    niter = -(-e // (nw * iter_edges))
    nblk = 2 * niter
    chunks_pw = nblk * _BLK
    epw = chunks_pw * _CHUNK
    pad = nw * epw - e
    acc_rows = -(-(n + 1) // (_NS * 8)) * (_NS * 8)

    padr = jnp.arange(pad, dtype=jnp.int32) % n
    ei = jnp.concatenate([edge_index, jnp.stack([padr, padr])], axis=1)
    idx = ei.reshape(2, nw, nblk, _BLK, _CHUNK)

    bm = 400 if n % 400 == 0 else 8
    grid = -(-n // bm)
    s0 = pl.pallas_call(
        _make_mm1_body(bm, pad, n),
        grid=(grid,),
        in_specs=[
            pl.BlockSpec((bm, din), lambda i: (i, 0)),
            pl.BlockSpec((din, dout), lambda i: (0, 0)),
            pl.BlockSpec((1, dout), lambda i: (0, 0)),
        ],
        out_specs=pl.BlockSpec((bm, dout), lambda i: (i, 0)),
        out_shape=jax.ShapeDtypeStruct((n, dout), jnp.float32),
    )(x, W, b.reshape(1, dout))

    parts = _build_sc_agg(n, din, nblk, acc_rows)(x, idx)

    out = pl.pallas_call(
        _mm2_body,
        grid=(grid,),
        in_specs=[
            pl.BlockSpec((bm, dout), lambda i: (i, 0)),
            pl.BlockSpec((_NC, bm, din), lambda i: (0, i, 0)),
            pl.BlockSpec((din, dout), lambda i: (0, 0)),
        ],
        out_specs=pl.BlockSpec((bm, dout), lambda i: (i, 0)),
        out_shape=jax.ShapeDtypeStruct((n, dout), jnp.float32),
    )(s0, parts, W)
    return out


# speedup vs baseline: 1.0180x; 1.0180x over previous
import functools

import jax
import jax.numpy as jnp
from jax import lax
from jax.experimental import pallas as pl
from jax.experimental.pallas import tpu as pltpu
from jax.experimental.pallas import tpu_sc as plsc

_SELFW = 1.0
_NC = 2
_NS = 16
_CHUNK = 120
_BLK = 3
_NBUF = 3


@functools.lru_cache(maxsize=None)
def _build_sc_agg(n, din, nblk, acc_rows):
    rows_per_tile = acc_rows // _NS
    assert rows_per_tile * _NS == acc_rows and nblk % 2 == 0
    assert (4 * _BLK) % _NBUF == 0
    nu = nblk // 2

    mesh = plsc.VectorSubcoreMesh(core_axis_name="c", subcore_axis_name="s")

    @functools.partial(
        pl.kernel,
        mesh=mesh,
        out_type=jax.ShapeDtypeStruct((_NC, acc_rows, din), jnp.float32),
        scratch_types=[
            pltpu.VMEM_SHARED((acc_rows, din), jnp.float32),
            pltpu.VMEM((_CHUNK, din), jnp.float32),
            pltpu.VMEM((_CHUNK, din), jnp.float32),
            pltpu.VMEM((_CHUNK, din), jnp.float32),
            pltpu.VMEM((2, _BLK, _CHUNK), jnp.int32),
            pltpu.VMEM((2, _BLK, _CHUNK), jnp.int32),
            pltpu.SemaphoreType.DMA,
            pltpu.SemaphoreType.DMA,
            pltpu.SemaphoreType.DMA,
            pltpu.SemaphoreType.DMA,
            pltpu.SemaphoreType.DMA,
        ],
    )
    def agg(x_hbm, idx_hbm, parts_hbm, acc, rows0, rows1, rows2, ib_a, ib_b,
            sem0, sem1, sem2, isem_a, isem_b):
        c = lax.axis_index("c")
        s = lax.axis_index("s")
        wid = s * _NC + c
        rows = (rows0, rows1, rows2)
        sems = (sem0, sem1, sem2)

        zeros16 = jnp.zeros((16,), jnp.float32)

        def zbody(i, carry):
            for j in range(din // 16):
                rows0[i, pl.ds(j * 16, 16)] = zeros16
            return carry

        lax.fori_loop(0, _CHUNK, zbody, 0)
        nfull, rem = divmod(rows_per_tile, _CHUNK)
        for j in range(nfull):
            r0 = s * rows_per_tile + j * _CHUNK
            pltpu.sync_copy(rows0, acc.at[pl.ds(r0, _CHUNK)])
        if rem:
            r0 = s * rows_per_tile + nfull * _CHUNK
            pltpu.sync_copy(rows0.at[pl.ds(0, rem)], acc.at[pl.ds(r0, rem)])
        plsc.subcore_barrier()

        def gstart(ib, v, g):
            j, d = divmod(v, 2)
            pltpu.async_copy(x_hbm.at[ib.at[d, j]], rows[g % _NBUF],
                             sems[g % _NBUF])

        def gwait(ib, v, g):
            j, d = divmod(v, 2)
            pltpu.make_async_copy(x_hbm.at[ib.at[d, j]], rows[g % _NBUF],
                                  sems[g % _NBUF]).wait()

        pltpu.sync_copy(idx_hbm.at[:, wid, 0], ib_a)
        pltpu.async_copy(idx_hbm.at[:, wid, 1], ib_b, isem_b)
        gstart(ib_a, 0, 0)
        gstart(ib_a, 1, 1)

        nv = 2 * _BLK

        def do_block(u, ib, nxt, nxt_isem, guarded):
            base = nv if guarded else 0
            for v in range(nv):
                gwait(ib, v, base + v)
                t = v + 2
                if t < nv:
                    gstart(ib, t, base + t)
                else:
                    def rollover(t=t):
                        if t == nv:
                            pltpu.make_async_copy(idx_hbm.at[:, wid, 0], nxt,
                                                  nxt_isem).wait()
                        gstart(nxt, t - nv, base + t)

                    if guarded:
                        pl.when(u < nu - 1)(rollover)
                    else:
                        rollover()
                j, d = divmod(v, 2)
                pltpu.sync_copy(rows[(base + v) % _NBUF], acc.at[ib.at[1 - d, j]],
                                add=True)

        def ebody(u, carry):
            do_block(u, ib_a, ib_b, isem_b, False)

            @pl.when(u < nu - 1)
            def _():
                pltpu.async_copy(idx_hbm.at[:, wid, 2 * u + 2], ib_a, isem_a)

            do_block(u, ib_b, ib_a, isem_a, True)

            @pl.when(u < nu - 1)
            def _():
                pltpu.async_copy(idx_hbm.at[:, wid, 2 * u + 3], ib_b, isem_b)

            return carry

        lax.fori_loop(0, nu, ebody, 0)
        plsc.subcore_barrier()

        for j in range(nfull):
            r0 = s * rows_per_tile + j * _CHUNK
            pltpu.sync_copy(acc.at[pl.ds(r0, _CHUNK)], rows0)
            pltpu.sync_copy(rows0, parts_hbm.at[c, pl.ds(r0, _CHUNK)])
        if rem:
            r0 = s * rows_per_tile + nfull * _CHUNK
            pltpu.sync_copy(acc.at[pl.ds(r0, rem)], rows0.at[pl.ds(0, rem)])
            pltpu.sync_copy(rows0.at[pl.ds(0, rem)],
                            parts_hbm.at[c, pl.ds(r0, rem)])

    return agg


def _make_mm_body(bm, pad, n):
    full = pad // n
    rem = pad % n

    def _mm_body(x_ref, p_ref, w_ref, b_ref, o_ref):
        rowid = bm * pl.program_id(0) + jax.lax.broadcasted_iota(
            jnp.int32, (bm, 1), 0)
        cnt = full + (rowid < rem).astype(jnp.float32)
        y = (_SELFW - 2.0 * cnt) * x_ref[...] + p_ref[0] + p_ref[1]
        o_ref[...] = jnp.dot(
            y, w_ref[...], preferred_element_type=jnp.float32
        ) + (2.0 + _SELFW) * b_ref[...]

    return _mm_body


def kernel(x, edge_index, W, b):
    n, din = x.shape
    dout = W.shape[1]
    e = edge_index.shape[1]
    nw = _NC * _NS

    iter_edges = 2 * _BLK * _CHUNK
    niter = -(-e // (nw * iter_edges))
    nblk = 2 * niter
    chunks_pw = nblk * _BLK
    epw = chunks_pw * _CHUNK
    pad = nw * epw - e
    acc_rows = -(-(n + 1) // (_NS * 8)) * (_NS * 8)

    padr = jnp.arange(pad, dtype=jnp.int32) % n
    ei = jnp.concatenate([edge_index, jnp.stack([padr, padr])], axis=1)
    idx = ei.reshape(2, nw, nblk, _BLK, _CHUNK)

    parts = _build_sc_agg(n, din, nblk, acc_rows)(x, idx)

    bm = 400 if n % 400 == 0 else 8
    grid = -(-n // bm)
    out = pl.pallas_call(
        _make_mm_body(bm, pad, n),
        grid=(grid,),
        in_specs=[
            pl.BlockSpec((bm, din), lambda i: (i, 0)),
            pl.BlockSpec((_NC, bm, din), lambda i: (0, i, 0)),
            pl.BlockSpec((din, dout), lambda i: (0, 0)),
            pl.BlockSpec((1, dout), lambda i: (0, 0)),
        ],
        out_specs=pl.BlockSpec((bm, dout), lambda i: (i, 0)),
        out_shape=jax.ShapeDtypeStruct((n, dout), jnp.float32),
    )(x, parts, W, b.reshape(1, dout))
    return out
